# Initial kernel scaffold; baseline (speedup 1.0000x reference)
import jax
import jax.numpy as jnp
from jax.experimental import pallas as pl


def kernel(coord, edge_index, target, Wzz_w, Wzz_b, W0, b0, W1, b1, W2, b2, M0w, M0b, M1w, M1b, M2w, M2b):
    raise NotImplementedError("write your pallas kernel here")


# speedup vs baseline: 4.6918x; 4.6918x over previous
import functools

import jax
import jax.numpy as jnp
from jax import lax
from jax.experimental import pallas as pl
from jax.experimental.pallas import tpu as pltpu
from jax.experimental.pallas import tpu_sc as plsc

N = 10000
E = 320000
DIM = 128
B = 8
NPG = N // B
EPG = E // B
DELTA = 0.1

NC, NS = 2, 16
NSC_NODES = N // NC
NSC_EDGES = E // NC
TILE_EDGES = NSC_EDGES // NS
CH = 128
NCHUNK = -(-TILE_EDGES // CH)
TILE_EDGES_PAD = NCHUNK * CH
DUMP = NSC_NODES
ACC_ROWS = NSC_NODES + 8


def _zero_acc(z128, acc, s, extra=None):
    r0 = pl.multiple_of(s * 312, 8)
    pltpu.sync_copy(z128.at[pl.ds(r0, 312)], acc.at[pl.ds(r0, 312)])

    @pl.when(s == 0)
    def _():
        pltpu.sync_copy(z128.at[pl.ds(4992, 8)], acc.at[pl.ds(4992, 8)])
    return r0


def _copy_out(acc, out, s, node0, r0):
    g0 = pl.multiple_of(node0 + r0, 8)
    pltpu.sync_copy(acc.at[pl.ds(r0, 312)], out.at[pl.ds(g0, 312)])

    @pl.when(s == 0)
    def _():
        t0 = pl.multiple_of(node0 + 4992, 8)
        pltpu.sync_copy(acc.at[pl.ds(4992, 8)], out.at[pl.ds(t0, 8)])


def _sc_agg_body(h, src_h, dst_h, z128, agg_out, acc, srcv, dstv, buf, sem):
    c = lax.axis_index("c")
    s = lax.axis_index("s")
    t = c * NS + s
    node0 = c * NSC_NODES

    pltpu.sync_copy(src_h.at[t], srcv)
    pltpu.sync_copy(dst_h.at[t], dstv)
    r0 = _zero_acc(z128, acc, s)
    plsc.subcore_barrier()

    def chunk(j, carry):
        pltpu.async_copy(h.at[srcv.at[j]], buf, sem).wait()
        pltpu.sync_copy(buf, acc.at[dstv.at[j]], add=True)
        return carry
    lax.fori_loop(0, NCHUNK, chunk, 0)
    plsc.subcore_barrier()
    _copy_out(acc, agg_out, s, node0, r0)


def _sc_cnt_body(dst_h, z128, ones_h, cnt_out, cntacc, dstv, onesv):
    c = lax.axis_index("c")
    s = lax.axis_index("s")
    t = c * NS + s
    node0 = c * NSC_NODES

    pltpu.sync_copy(dst_h.at[t], dstv)
    pltpu.sync_copy(ones_h, onesv)
    r0 = _zero_acc(z128, cntacc, s)
    plsc.subcore_barrier()

    def chunk(j, carry):
        pltpu.sync_copy(onesv, cntacc.at[dstv.at[j]], add=True)
        return carry
    lax.fori_loop(0, NCHUNK, chunk, 0)
    plsc.subcore_barrier()
    _copy_out(cntacc, cnt_out, s, node0, r0)


@functools.lru_cache(maxsize=None)
def _sc_kernels():
    mesh = plsc.VectorSubcoreMesh(core_axis_name="c", subcore_axis_name="s",
                                  num_cores=NC, num_subcores=NS)
    cnt_k = functools.partial(
        pl.kernel,
        out_type=jax.ShapeDtypeStruct((N, DIM), jnp.float32),
        mesh=mesh,
        scratch_types=[
            pltpu.VMEM_SHARED((ACC_ROWS, DIM), jnp.float32),
            pltpu.VMEM((NCHUNK, CH), jnp.int32),
            pltpu.VMEM((CH, DIM), jnp.float32),
        ],
    )(_sc_cnt_body)
    agg = functools.partial(
        pl.kernel,
        out_type=jax.ShapeDtypeStruct((N, DIM), jnp.float32),
        mesh=mesh,
        scratch_types=[
            pltpu.VMEM_SHARED((ACC_ROWS, DIM), jnp.float32),
            pltpu.VMEM((NCHUNK, CH), jnp.int32),
            pltpu.VMEM((NCHUNK, CH), jnp.int32),
            pltpu.VMEM((CH, DIM), jnp.float32),
            pltpu.SemaphoreType.DMA,
        ],
    )(_sc_agg_body)
    return cnt_k, agg


def _h0_body(cx, cy, w0, w1, bz, out):
    out[...] = cx[...] * w0[...] + cy[...] * w1[...] + bz[...]


def _upd_body(h_ref, agg_ref, cnt_ref, w_ref, b_ref, out_ref):
    cnt = jnp.max(cnt_ref[...], axis=1, keepdims=True)
    cnt = jnp.maximum(cnt, 1.0)
    a = agg_ref[...] / cnt
    out_ref[...] = h_ref[...] + DELTA * jnp.maximum(
        jnp.dot(a, w_ref[...], preferred_element_type=jnp.float32)
        + b_ref[...], 0.0)


def _head_body(h_ref, m0w, m0b, m1w, m1b, m2w, m2b, tgt, out_ref):
    hh = h_ref[...]
    rows = lax.broadcasted_iota(jnp.int32, (B, N), 0)
    cols = lax.broadcasted_iota(jnp.int32, (B, N), 1)
    sel = jnp.where(cols // NPG == rows, 1.0 / NPG, 0.0)
    g = jnp.dot(sel, hh, preferred_element_type=jnp.float32)
    y = jnp.maximum(jnp.dot(g, m0w[...],
                            preferred_element_type=jnp.float32) + m0b[...], 0.0)
    y = jnp.maximum(jnp.dot(y, m1w[...],
                            preferred_element_type=jnp.float32) + m1b[...], 0.0)
    y = jnp.dot(y, m2w[...], preferred_element_type=jnp.float32) + m2b[...]
    pr = lax.broadcasted_iota(jnp.int32, (B // 2, B), 0)
    pc = lax.broadcasted_iota(jnp.int32, (B // 2, B), 1)
    pair = (jnp.where(pc == 2 * pr, 1.0, 0.0)
            - jnp.where(pc == 2 * pr + 1, 1.0, 0.0))
    d = jnp.dot(pair, y, preferred_element_type=jnp.float32)
    lossv = jnp.maximum(0.0, -tgt[...] * d)
    out_ref[...] = jnp.mean(lossv).reshape(1, 1)


_RB = 2000


def _tc_h0(cx, cy, w0, w1, bz):
    return pl.pallas_call(
        _h0_body,
        grid=(N // _RB,),
        in_specs=[
            pl.BlockSpec((_RB, 1), lambda i: (i, 0)),
            pl.BlockSpec((_RB, 1), lambda i: (i, 0)),
            pl.BlockSpec((1, DIM), lambda i: (0, 0)),
            pl.BlockSpec((1, DIM), lambda i: (0, 0)),
            pl.BlockSpec((1, DIM), lambda i: (0, 0)),
        ],
        out_specs=pl.BlockSpec((_RB, DIM), lambda i: (i, 0)),
        out_shape=jax.ShapeDtypeStruct((N, DIM), jnp.float32),
    )(cx, cy, w0, w1, bz)


def _tc_upd(h, agg, cnt, w, b):
    return pl.pallas_call(
        _upd_body,
        grid=(N // _RB,),
        in_specs=[
            pl.BlockSpec((_RB, DIM), lambda i: (i, 0)),
            pl.BlockSpec((_RB, DIM), lambda i: (i, 0)),
            pl.BlockSpec((_RB, DIM), lambda i: (i, 0)),
            pl.BlockSpec((DIM, DIM), lambda i: (0, 0)),
            pl.BlockSpec((1, DIM), lambda i: (0, 0)),
        ],
        out_specs=pl.BlockSpec((_RB, DIM), lambda i: (i, 0)),
        out_shape=jax.ShapeDtypeStruct((N, DIM), jnp.float32),
    )(h, agg, cnt, w, b)


def _tc_head(h, m0w, m0b, m1w, m1b, m2w, m2b, tgt):
    return pl.pallas_call(
        _head_body,
        out_shape=jax.ShapeDtypeStruct((1, 1), jnp.float32),
    )(h, m0w, m0b, m1w, m1b, m2w, m2b, tgt)


def kernel(coord, edge_index, target, Wzz_w, Wzz_b, W0, b0, W1, b1, W2, b2,
           M0w, M0b, M1w, M1b, M2w, M2b):
    src = edge_index[0]
    dst = edge_index[1]
    sc_base = (jnp.arange(E, dtype=jnp.int32) // NSC_EDGES) * NSC_NODES
    dstl = dst - sc_base
    pad = TILE_EDGES_PAD - TILE_EDGES
    src_p = jnp.pad(src.reshape(NC * NS, TILE_EDGES),
                    ((0, 0), (0, pad))).reshape(NC * NS, NCHUNK, CH)
    dst_p = jnp.pad(dstl.reshape(NC * NS, TILE_EDGES),
                    ((0, 0), (0, pad)),
                    constant_values=DUMP).reshape(NC * NS, NCHUNK, CH)
    z128 = jnp.zeros((NSC_NODES, DIM), jnp.float32)
    ones128 = jnp.ones((CH, DIM), jnp.float32)

    sc_cnt, sc_agg = _sc_kernels()
    h = _tc_h0(coord[:, 0:1], coord[:, 1:2],
               Wzz_w[0:1, :], Wzz_w[1:2, :], Wzz_b.reshape(1, DIM))
    cnt = sc_cnt(dst_p, z128, ones128)
    agg = sc_agg(h, src_p, dst_p, z128)
    h = _tc_upd(h, agg, cnt, W0, b0.reshape(1, DIM))
    agg = sc_agg(h, src_p, dst_p, z128)
    h = _tc_upd(h, agg, cnt, W1, b1.reshape(1, DIM))
    agg = sc_agg(h, src_p, dst_p, z128)
    h = _tc_upd(h, agg, cnt, W2, b2.reshape(1, DIM))
    loss = _tc_head(h, M0w, M0b.reshape(1, -1), M1w, M1b.reshape(1, -1),
                    M2w, M2b.reshape(1, 1), target.reshape(B // 2, 1))
    return loss[0, 0]
